# baseline (speedup 1.0000x reference)
import functools

import jax
import jax.numpy as jnp
from jax import lax
from jax.experimental import pallas as pl
from jax.experimental.pallas import tpu as pltpu
from jax.experimental.pallas import tpu_sc as plsc

_N = 10000
_E = 320000
_D_IN = 128
_D_HID = 4
_D_OUT = 2

_NC = 2
_NS = 16
_NW = _NC * _NS

_E_PER = 10112
_E_PAD = _NW * _E_PER
_NGRP = _E_PER // 16

_NP = 10240


_sc_mesh = plsc.VectorSubcoreMesh(core_axis_name="c", subcore_axis_name="s")


def _zero_fill(buf, n):
    z = jnp.zeros((16,), jnp.float32)

    def body(i, _):
        for u in range(8):
            buf[pl.ds(128 * i + 16 * u, 16)] = z
        return None

    lax.fori_loop(0, n // 128, body, None)


@functools.partial(
    pl.kernel,
    out_type=jax.ShapeDtypeStruct((_NW, _NP), jnp.float32),
    mesh=_sc_mesh,
    scratch_types=[
        pltpu.VMEM((_E_PER,), jnp.int32),
        pltpu.VMEM((_NP,), jnp.float32),
    ],
    compiler_params=pltpu.CompilerParams(needs_layout_passes=False),
)
def _deg_kernel(dst_hbm, out_hbm, dst_v, acc_v):
    cid = lax.axis_index("c")
    sid = lax.axis_index("s")
    w = sid * _NC + cid
    pltpu.sync_copy(dst_hbm.at[w], dst_v)
    _zero_fill(acc_v, _NP)
    ones = jnp.ones((16,), jnp.float32)

    def body(t, _):
        didx = dst_v[pl.ds(16 * t, 16)]
        plsc.addupdate_scatter(acc_v, [didx], ones)
        return None

    lax.fori_loop(0, _NGRP, body, None)
    pltpu.sync_copy(acc_v, out_hbm.at[w])


@functools.partial(
    pl.kernel,
    out_type=jax.ShapeDtypeStruct((_NW, _D_HID * _NP), jnp.float32),
    mesh=_sc_mesh,
    scratch_types=[
        pltpu.VMEM((_E_PER,), jnp.int32),
        pltpu.VMEM((_E_PER,), jnp.int32),
        pltpu.VMEM((_D_HID * _NP,), jnp.float32),
        pltpu.VMEM((_D_HID * _NP,), jnp.float32),
    ],
    compiler_params=pltpu.CompilerParams(needs_layout_passes=False),
)
def _agg_kernel(src_hbm, dst_hbm, g_hbm, out_hbm, src_v, dst_v, g_v, acc_v):
    cid = lax.axis_index("c")
    sid = lax.axis_index("s")
    w = sid * _NC + cid
    pltpu.sync_copy(src_hbm.at[w], src_v)
    pltpu.sync_copy(dst_hbm.at[w], dst_v)
    for c in range(_D_HID):
        pltpu.sync_copy(g_hbm.at[c], g_v.at[pl.ds(c * _NP, _NP)])
    _zero_fill(acc_v, _D_HID * _NP)

    def body(t, _):
        sl = pl.ds(16 * t, 16)
        idx = src_v[sl]
        didx = dst_v[sl]
        for c in range(_D_HID):
            v = plsc.load_gather(g_v, [idx + c * _NP])
            plsc.addupdate_scatter(acc_v, [didx + c * _NP], v)
        return None

    lax.fori_loop(0, _NGRP, body, None)
    pltpu.sync_copy(acc_v, out_hbm.at[w])


def _tc1_body(x_ref, w1_ref, deg_ref, dinv_ref, g1_ref):
    deg = jnp.sum(deg_ref[...], axis=0, keepdims=True) + 1.0
    r = lax.rsqrt(deg)
    dinv = r * (1.5 - 0.5 * deg * r * r)
    dinv_ref[...] = dinv
    h0 = lax.dot_general(w1_ref[...], x_ref[...], (((0,), (1,)), ((), ())),
                         preferred_element_type=jnp.float32,
                         precision=lax.Precision.HIGHEST)
    g1_ref[...] = h0 * dinv


_tc1 = pl.pallas_call(
    _tc1_body,
    out_shape=[
        jax.ShapeDtypeStruct((1, _NP), jnp.float32),
        jax.ShapeDtypeStruct((_D_HID, _NP), jnp.float32),
    ],
)


def _tc2_body(t1_ref, g1_ref, dinv_ref, b1_ref, g2_ref):
    dinv = dinv_ref[...]
    t1 = jnp.sum(t1_ref[...], axis=0) + g1_ref[...]
    h = jnp.maximum(dinv * t1 + b1_ref[...], 0.0)
    g2_ref[...] = dinv * h


_tc2 = pl.pallas_call(
    _tc2_body,
    out_shape=jax.ShapeDtypeStruct((_D_HID, _NP), jnp.float32),
)


def _tc3_body(t2_ref, g2_ref, dinv_ref, wmu_ref, bmu_ref,
              wls_ref, bls_ref, mu_ref, ls_ref):
    hag = dinv_ref[...] * (jnp.sum(t2_ref[...], axis=0) + g2_ref[...])
    mu_ref[...] = lax.dot_general(wmu_ref[...], hag, (((0,), (0,)), ((), ())),
                                  preferred_element_type=jnp.float32,
                                  precision=lax.Precision.HIGHEST) + bmu_ref[...]
    ls_ref[...] = lax.dot_general(wls_ref[...], hag, (((0,), (0,)), ((), ())),
                                  preferred_element_type=jnp.float32,
                                  precision=lax.Precision.HIGHEST) + bls_ref[...]


_tc3 = pl.pallas_call(
    _tc3_body,
    out_shape=[
        jax.ShapeDtypeStruct((_D_OUT, _NP), jnp.float32),
        jax.ShapeDtypeStruct((_D_OUT, _NP), jnp.float32),
    ],
)


def kernel(x, edge_index, W1, b1, Wmu, bmu, Wls, bls):
    src = edge_index[0]
    dst = edge_index[1]
    pad = _N + (jnp.arange(_E_PAD - _E, dtype=jnp.int32) % (_NP - _N))
    src2 = jnp.concatenate([src, pad]).reshape(_NW, _E_PER)
    dst2 = jnp.concatenate([dst, pad]).reshape(_NW, _E_PER)
    x_pad = jnp.zeros((_NP, _D_IN), jnp.float32).at[:_N].set(x)

    deg = _deg_kernel(dst2)
    dinv, g1 = _tc1(x_pad, W1, deg)
    t1 = _agg_kernel(src2, dst2, g1).reshape(_NW, _D_HID, _NP)
    g2 = _tc2(t1, g1, dinv, b1.reshape(_D_HID, 1))
    t2 = _agg_kernel(src2, dst2, g2).reshape(_NW, _D_HID, _NP)
    mu, ls = _tc3(t2, g2, dinv, Wmu, bmu.reshape(_D_OUT, 1),
                  Wls, bls.reshape(_D_OUT, 1))
    return mu[:, :_N].T, ls[:, :_N].T


# speedup vs baseline: 1.3749x; 1.3749x over previous
import functools

import jax
import jax.numpy as jnp
from jax import lax
from jax.experimental import pallas as pl
from jax.experimental.pallas import tpu as pltpu
from jax.experimental.pallas import tpu_sc as plsc

_N = 10000
_E = 320000
_D_IN = 128
_D_HID = 4
_D_OUT = 2

_NC = 2
_NS = 16
_NW = _NC * _NS

_E_PER = _E // _NW
_NGRP = _E_PER // 16

_NP = 10240


_sc_mesh = plsc.VectorSubcoreMesh(core_axis_name="c", subcore_axis_name="s")


def _zero_fill(buf, n):
    z = jnp.zeros((16,), jnp.float32)

    def body(i, _):
        for u in range(8):
            buf[pl.ds(128 * i + 16 * u, 16)] = z
        return None

    lax.fori_loop(0, n // 128, body, None)


@functools.partial(
    pl.kernel,
    out_type=jax.ShapeDtypeStruct((_NW, _NP), jnp.float32),
    mesh=_sc_mesh,
    scratch_types=[
        pltpu.VMEM((_E_PER,), jnp.int32),
        pltpu.VMEM((_NP,), jnp.float32),
    ],
    compiler_params=pltpu.CompilerParams(needs_layout_passes=False),
)
def _deg_kernel(dst_hbm, out_hbm, dst_v, acc_v):
    cid = lax.axis_index("c")
    sid = lax.axis_index("s")
    w = sid * _NC + cid
    pltpu.sync_copy(dst_hbm.at[pl.ds(w * _E_PER, _E_PER)], dst_v)
    _zero_fill(acc_v, _NP)
    ones = jnp.ones((16,), jnp.float32)

    @plsc.parallel_loop(0, _NGRP, 1, unroll=4)
    def _(t):
        didx = dst_v[pl.ds(16 * t, 16)]
        plsc.addupdate_scatter(acc_v, [didx], ones)

    pltpu.sync_copy(acc_v, out_hbm.at[w])


@functools.partial(
    pl.kernel,
    out_type=jax.ShapeDtypeStruct((_NW, _D_HID * _NP), jnp.float32),
    mesh=_sc_mesh,
    scratch_types=[
        pltpu.VMEM((_E_PER,), jnp.int32),
        pltpu.VMEM((_E_PER,), jnp.int32),
        pltpu.VMEM((_D_HID * _NP,), jnp.float32),
        pltpu.VMEM((_D_HID * _NP,), jnp.float32),
    ],
    compiler_params=pltpu.CompilerParams(needs_layout_passes=False),
)
def _agg_kernel(src_hbm, dst_hbm, g_hbm, out_hbm, src_v, dst_v, g_v, acc_v):
    cid = lax.axis_index("c")
    sid = lax.axis_index("s")
    w = sid * _NC + cid
    pltpu.sync_copy(src_hbm.at[pl.ds(w * _E_PER, _E_PER)], src_v)
    pltpu.sync_copy(dst_hbm.at[pl.ds(w * _E_PER, _E_PER)], dst_v)
    for c in range(_D_HID):
        pltpu.sync_copy(g_hbm.at[c], g_v.at[pl.ds(c * _NP, _NP)])
    _zero_fill(acc_v, _D_HID * _NP)

    @plsc.parallel_loop(0, _NGRP, 1, unroll=2)
    def _(t):
        sl = pl.ds(16 * t, 16)
        idx = src_v[sl]
        didx = dst_v[sl]
        for c in range(_D_HID):
            v = plsc.load_gather(g_v, [idx + c * _NP])
            plsc.addupdate_scatter(acc_v, [didx + c * _NP], v)

    pltpu.sync_copy(acc_v, out_hbm.at[w])


def _tc1_body(x_ref, w1_ref, deg_ref, dinv_ref, g1_ref):
    deg = jnp.sum(deg_ref[...], axis=0, keepdims=True) + 1.0
    r = lax.rsqrt(deg)
    dinv = r * (1.5 - 0.5 * deg * r * r)
    dinv_ref[...] = dinv
    h0 = lax.dot_general(w1_ref[...], x_ref[...], (((0,), (1,)), ((), ())),
                         preferred_element_type=jnp.float32,
                         precision=lax.Precision.HIGHEST)
    g1_ref[...] = jnp.pad(h0, ((0, 0), (0, _NP - _N))) * dinv


_tc1 = pl.pallas_call(
    _tc1_body,
    out_shape=[
        jax.ShapeDtypeStruct((1, _NP), jnp.float32),
        jax.ShapeDtypeStruct((_D_HID, _NP), jnp.float32),
    ],
)


def _tc2_body(t1_ref, g1_ref, dinv_ref, b1_ref, g2_ref):
    dinv = dinv_ref[...]
    t1 = jnp.sum(t1_ref[...], axis=0).reshape(_D_HID, _NP) + g1_ref[...]
    h = jnp.maximum(dinv * t1 + b1_ref[...], 0.0)
    g2_ref[...] = dinv * h


_tc2 = pl.pallas_call(
    _tc2_body,
    out_shape=jax.ShapeDtypeStruct((_D_HID, _NP), jnp.float32),
)


def _tc3_body(t2_ref, g2_ref, dinv_ref, wmu_ref, bmu_ref,
              wls_ref, bls_ref, mu_ref, ls_ref):
    t2 = jnp.sum(t2_ref[...], axis=0).reshape(_D_HID, _NP)
    hag = dinv_ref[...] * (t2 + g2_ref[...])
    mu_ref[...] = lax.dot_general(wmu_ref[...], hag, (((0,), (0,)), ((), ())),
                                  preferred_element_type=jnp.float32,
                                  precision=lax.Precision.HIGHEST) + bmu_ref[...]
    ls_ref[...] = lax.dot_general(wls_ref[...], hag, (((0,), (0,)), ((), ())),
                                  preferred_element_type=jnp.float32,
                                  precision=lax.Precision.HIGHEST) + bls_ref[...]


_tc3 = pl.pallas_call(
    _tc3_body,
    out_shape=[
        jax.ShapeDtypeStruct((_D_OUT, _NP), jnp.float32),
        jax.ShapeDtypeStruct((_D_OUT, _NP), jnp.float32),
    ],
)


def kernel(x, edge_index, W1, b1, Wmu, bmu, Wls, bls):
    src = edge_index[0]
    dst = edge_index[1]
    deg = _deg_kernel(dst)
    dinv, g1 = _tc1(x, W1, deg)
    t1 = _agg_kernel(src, dst, g1)
    g2 = _tc2(t1, g1, dinv, b1.reshape(_D_HID, 1))
    t2 = _agg_kernel(src, dst, g2)
    mu, ls = _tc3(t2, g2, dinv, Wmu, bmu.reshape(_D_OUT, 1),
                  Wls, bls.reshape(_D_OUT, 1))
    return mu[:, :_N].T, ls[:, :_N].T


# speedup vs baseline: 1.5704x; 1.1422x over previous
import functools

import jax
import jax.numpy as jnp
from jax import lax
from jax.experimental import pallas as pl
from jax.experimental.pallas import tpu as pltpu
from jax.experimental.pallas import tpu_sc as plsc

_N = 10000
_E = 320000
_D_IN = 128
_D_HID = 4
_D_OUT = 2

_NC = 2
_NS = 16
_NW = _NC * _NS

_NT = _E // 128
_T_PER = _NT // _NW
_T_REM = _NT - _NW * _T_PER

_NP = 10240


_sc_mesh = plsc.VectorSubcoreMesh(core_axis_name="c", subcore_axis_name="s")


def _zero_fill(buf, n):
    z = jnp.zeros((16,), jnp.float32)

    def body(i, _):
        for u in range(8):
            buf[pl.ds(128 * i + 16 * u, 16)] = z
        return None

    lax.fori_loop(0, n // 128, body, None)


def _load_edge_tiles(ei_hbm, ei_v, w):
    tlo = w * _T_PER + jnp.minimum(w, _T_REM)
    cnt = _T_PER + jnp.where(w < _T_REM, 1, 0)
    pltpu.sync_copy(ei_hbm.at[pl.ds(tlo, _T_PER)], ei_v.at[pl.ds(0, _T_PER)])

    @pl.when(w < _T_REM)
    def _():
        pltpu.sync_copy(ei_hbm.at[pl.ds(tlo + _T_PER, 1)],
                        ei_v.at[pl.ds(_T_PER, 1)])

    return cnt


@functools.partial(
    pl.kernel,
    out_type=jax.ShapeDtypeStruct((_NW, _NP), jnp.float32),
    mesh=_sc_mesh,
    scratch_types=[
        pltpu.VMEM((_T_PER + 1, 2, 128), jnp.int32),
        pltpu.VMEM((_NP,), jnp.float32),
    ],
    compiler_params=pltpu.CompilerParams(needs_layout_passes=False),
)
def _deg_kernel(ei_hbm, out_hbm, ei_v, acc_v):
    cid = lax.axis_index("c")
    sid = lax.axis_index("s")
    w = sid * _NC + cid
    cnt = _load_edge_tiles(ei_hbm, ei_v, w)
    _zero_fill(acc_v, _NP)
    ones = jnp.ones((16,), jnp.float32)

    @plsc.parallel_loop(0, cnt, 1, unroll=2)
    def _(t):
        for k in range(8):
            didx = ei_v[t, 1, pl.ds(16 * k, 16)]
            plsc.addupdate_scatter(acc_v, [didx], ones)

    pltpu.sync_copy(acc_v, out_hbm.at[w])


@functools.partial(
    pl.kernel,
    out_type=jax.ShapeDtypeStruct((_NW, _D_HID * _NP), jnp.float32),
    mesh=_sc_mesh,
    scratch_types=[
        pltpu.VMEM((_T_PER + 1, 2, 128), jnp.int32),
        pltpu.VMEM((_D_HID * _NP,), jnp.float32),
        pltpu.VMEM((_D_HID * _NP,), jnp.float32),
    ],
    compiler_params=pltpu.CompilerParams(needs_layout_passes=False),
)
def _agg_kernel(ei_hbm, g_hbm, out_hbm, ei_v, g_v, acc_v):
    cid = lax.axis_index("c")
    sid = lax.axis_index("s")
    w = sid * _NC + cid
    cnt = _load_edge_tiles(ei_hbm, ei_v, w)
    for c in range(_D_HID):
        pltpu.sync_copy(g_hbm.at[c], g_v.at[pl.ds(c * _NP, _NP)])
    _zero_fill(acc_v, _D_HID * _NP)

    @plsc.parallel_loop(0, cnt, 1, unroll=2)
    def _(t):
        for k in range(8):
            sl = pl.ds(16 * k, 16)
            idx = ei_v[t, 0, sl]
            didx = ei_v[t, 1, sl]
            for c in range(_D_HID):
                v = plsc.load_gather(g_v, [idx + c * _NP])
                plsc.addupdate_scatter(acc_v, [didx + c * _NP], v)

    pltpu.sync_copy(acc_v, out_hbm.at[w])


def _tc1_body(x_ref, w1_ref, deg_ref, dinv_ref, g1_ref):
    deg = jnp.sum(deg_ref[...], axis=0, keepdims=True) + 1.0
    r = lax.rsqrt(deg)
    dinv = r * (1.5 - 0.5 * deg * r * r)
    dinv_ref[...] = dinv
    h0 = lax.dot_general(w1_ref[...], x_ref[...], (((0,), (1,)), ((), ())),
                         preferred_element_type=jnp.float32,
                         precision=lax.Precision.HIGHEST)
    g1_ref[...] = jnp.pad(h0, ((0, 0), (0, _NP - _N))) * dinv


_tc1 = pl.pallas_call(
    _tc1_body,
    out_shape=[
        jax.ShapeDtypeStruct((1, _NP), jnp.float32),
        jax.ShapeDtypeStruct((_D_HID, _NP), jnp.float32),
    ],
)


def _tc2_body(t1_ref, g1_ref, dinv_ref, b1_ref, g2_ref):
    dinv = dinv_ref[...]
    t1 = jnp.sum(t1_ref[...], axis=0).reshape(_D_HID, _NP) + g1_ref[...]
    h = jnp.maximum(dinv * t1 + b1_ref[...], 0.0)
    g2_ref[...] = dinv * h


_tc2 = pl.pallas_call(
    _tc2_body,
    out_shape=jax.ShapeDtypeStruct((_D_HID, _NP), jnp.float32),
)


def _tc3_body(t2_ref, g2_ref, dinv_ref, wmu_ref, bmu_ref,
              wls_ref, bls_ref, mu_ref, ls_ref):
    t2 = jnp.sum(t2_ref[...], axis=0).reshape(_D_HID, _NP)
    hag = dinv_ref[...] * (t2 + g2_ref[...])
    mu_ref[...] = lax.dot_general(wmu_ref[...], hag, (((0,), (0,)), ((), ())),
                                  preferred_element_type=jnp.float32,
                                  precision=lax.Precision.HIGHEST) + bmu_ref[...]
    ls_ref[...] = lax.dot_general(wls_ref[...], hag, (((0,), (0,)), ((), ())),
                                  preferred_element_type=jnp.float32,
                                  precision=lax.Precision.HIGHEST) + bls_ref[...]


_tc3 = pl.pallas_call(
    _tc3_body,
    out_shape=[
        jax.ShapeDtypeStruct((_D_OUT, _NP), jnp.float32),
        jax.ShapeDtypeStruct((_D_OUT, _NP), jnp.float32),
    ],
)


def kernel(x, edge_index, W1, b1, Wmu, bmu, Wls, bls):
    ei3 = jnp.swapaxes(edge_index.reshape(2, _NT, 128), 0, 1)
    deg = _deg_kernel(ei3)
    dinv, g1 = _tc1(x, W1, deg)
    t1 = _agg_kernel(ei3, g1)
    g2 = _tc2(t1, g1, dinv, b1.reshape(_D_HID, 1))
    t2 = _agg_kernel(ei3, g2)
    mu, ls = _tc3(t2, g2, dinv, Wmu, bmu.reshape(_D_OUT, 1),
                  Wls, bls.reshape(_D_OUT, 1))
    return mu[:, :_N].T, ls[:, :_N].T
